# Initial kernel scaffold; baseline (speedup 1.0000x reference)
import jax
import jax.numpy as jnp
from jax.experimental import pallas as pl


def kernel(h, s_l, W1, b1, W2, b2):
    raise NotImplementedError("write your pallas kernel here")


# speedup vs baseline: 5.8399x; 5.8399x over previous
import functools

import jax
import jax.numpy as jnp
import numpy as np
from jax import lax
from jax.experimental import pallas as pl
from jax.experimental.pallas import tpu as pltpu

_N = 10000
_HIDDEN = 256
_M = 5
_NUP = 2500
_NDOWN = _N - _NUP
_ROWS, _COLS = 8, 1280
_NPAD = _ROWS * _COLS


def _fps_mask_body(px_ref, py_ref, pz_ref, mask_ref, dists_ref):
    px = px_ref[...]
    py = py_ref[...]
    pz = pz_ref[...]
    row = lax.broadcasted_iota(jnp.int32, (_ROWS, _COLS), 0)
    col = lax.broadcasted_iota(jnp.int32, (_ROWS, _COLS), 1)
    gidx = row * _COLS + col
    valid = gidx < _N
    neg_inf = jnp.float32(-jnp.inf)
    dists_ref[...] = jnp.where(valid, jnp.float32(1e10), neg_inf)
    mask0 = gidx == 0
    mask_ref[...] = mask0.astype(jnp.int32)
    x0 = jnp.max(jnp.where(mask0, px, neg_inf))
    y0 = jnp.max(jnp.where(mask0, py, neg_inf))
    z0 = jnp.max(jnp.where(mask0, pz, neg_inf))

    def body(_, carry):
        x0, y0, z0 = carry
        dx = px - x0
        dy = py - y0
        dz = pz - z0
        d = (dx * dx + dy * dy) + dz * dz
        dists = jnp.minimum(dists_ref[...], d)
        dists_ref[...] = dists
        m = jnp.max(dists)
        sel = jnp.where(dists == m, gidx, jnp.int32(2**30))
        nxt = jnp.min(sel)
        eq = gidx == nxt
        mask_ref[...] = jnp.maximum(mask_ref[...], eq.astype(jnp.int32))
        x0 = jnp.max(jnp.where(eq, px, neg_inf))
        y0 = jnp.max(jnp.where(eq, py, neg_inf))
        z0 = jnp.max(jnp.where(eq, pz, neg_inf))
        return (x0, y0, z0)

    lax.fori_loop(1, _NUP, body, (x0, y0, z0))


def _fps_mask(pos):
    def pad(c):
        return jnp.zeros((_NPAD,), jnp.float32).at[:_N].set(c).reshape(_ROWS, _COLS)

    pxp, pyp, pzp = pad(pos[:, 0]), pad(pos[:, 1]), pad(pos[:, 2])
    mask = pl.pallas_call(
        _fps_mask_body,
        out_shape=jax.ShapeDtypeStruct((_ROWS, _COLS), jnp.int32),
        scratch_shapes=[pltpu.VMEM((_ROWS, _COLS), jnp.float32)],
    )(pxp, pyp, pzp)
    return mask.reshape(-1)[:_N].astype(bool)


def kernel(h, s_l, W1, b1, W2, b2):
    n = h.shape[0]
    pos = lax.stop_gradient(s_l)
    up_mask = _fps_mask(pos)

    up_idx = jnp.argsort(~up_mask, stable=True)[:_NUP].astype(jnp.int32)
    down_idx = jnp.argsort(up_mask, stable=True)[:_NDOWN].astype(jnp.int32)
    pd = pos[down_idx]
    pu = pos[up_idx]
    d2 = (jnp.sum(pd**2, axis=1)[:, None]
          + jnp.sum(pu**2, axis=1)[None, :]
          - 2.0 * pd @ pu.T)
    _, nbr = jax.lax.top_k(-d2, _M)
    j = up_idx[nbr].reshape(-1)
    i = jnp.repeat(down_idx, _M)

    feats = jnp.concatenate([h, s_l], axis=1) @ W1 + b1
    msg = jax.nn.relu((feats[i] - feats[j]) @ W2 + b2)
    agg = jax.ops.segment_max(msg, j, num_segments=n)
    agg = jnp.where(jnp.isfinite(agg), agg, 0.0)
    out = feats + agg
    return out, up_mask, i, j


# speedup vs baseline: 6.7072x; 1.1485x over previous
import functools

import jax
import jax.numpy as jnp
import numpy as np
from jax import lax
from jax.experimental import pallas as pl
from jax.experimental.pallas import tpu as pltpu

_N = 10000
_HIDDEN = 256
_M = 5
_NUP = 2500
_NDOWN = _N - _NUP
_ROWS, _COLS = 8, 1280
_NPAD = _ROWS * _COLS


def _fps_mask_body(px_ref, py_ref, pz_ref, mask_ref, dists_ref):
    px = px_ref[...]
    py = py_ref[...]
    pz = pz_ref[...]
    row = lax.broadcasted_iota(jnp.int32, (_ROWS, _COLS), 0)
    col = lax.broadcasted_iota(jnp.int32, (_ROWS, _COLS), 1)
    gidx = row * _COLS + col
    valid = gidx < _N
    neg_inf = jnp.float32(-jnp.inf)
    dists_ref[...] = jnp.where(valid, jnp.float32(1e10), neg_inf)
    mask0 = gidx == 0
    mask_ref[...] = mask0.astype(jnp.int32)
    x0 = jnp.max(jnp.where(mask0, px, neg_inf))
    y0 = jnp.max(jnp.where(mask0, py, neg_inf))
    z0 = jnp.max(jnp.where(mask0, pz, neg_inf))

    def body(_, carry):
        x0, y0, z0 = carry
        dx = px - x0
        dy = py - y0
        dz = pz - z0
        d = (dx * dx + dy * dy) + dz * dz
        dists = jnp.minimum(dists_ref[...], d)
        dists_ref[...] = dists
        m = jnp.max(dists)
        sel = jnp.where(dists == m, gidx, jnp.int32(2**30))
        nxt = jnp.min(sel)
        eq = gidx == nxt
        mask_ref[...] = jnp.maximum(mask_ref[...], eq.astype(jnp.int32))
        x0 = jnp.max(jnp.where(eq, px, neg_inf))
        y0 = jnp.max(jnp.where(eq, py, neg_inf))
        z0 = jnp.max(jnp.where(eq, pz, neg_inf))
        return (x0, y0, z0)

    lax.fori_loop(1, _NUP, body, (x0, y0, z0))


def _fps_mask(pos):
    def pad(c):
        return jnp.zeros((_NPAD,), jnp.float32).at[:_N].set(c).reshape(_ROWS, _COLS)

    pxp, pyp, pzp = pad(pos[:, 0]), pad(pos[:, 1]), pad(pos[:, 2])
    mask = pl.pallas_call(
        _fps_mask_body,
        out_shape=jax.ShapeDtypeStruct((_ROWS, _COLS), jnp.int32),
        scratch_shapes=[pltpu.VMEM((_ROWS, _COLS), jnp.float32)],
    )(pxp, pyp, pzp)
    return mask.reshape(-1)[:_N].astype(bool)


_XCOLS = 384
_MMBLK = 1000
_KNNBLK = 768
_NDOWNPAD = 7680
_NUPPAD = 2560


def _dense_body(x_ref, w1_ref, b1_ref, w2_ref, b2_ref, feats_ref, g_ref):
    feats = jnp.dot(x_ref[...], w1_ref[...],
                    preferred_element_type=jnp.float32) + b1_ref[...]
    feats_ref[...] = feats
    g_ref[...] = jnp.dot(feats, w2_ref[...],
                         preferred_element_type=jnp.float32)


def _dense_feats_g(x, W1p, b1, W2, b2):
    grid = _N // _MMBLK
    return pl.pallas_call(
        _dense_body,
        grid=(grid,),
        in_specs=[
            pl.BlockSpec((_MMBLK, _XCOLS), lambda r: (r, 0)),
            pl.BlockSpec((_XCOLS, _HIDDEN), lambda r: (0, 0)),
            pl.BlockSpec((1, _HIDDEN), lambda r: (0, 0)),
            pl.BlockSpec((_HIDDEN, _HIDDEN), lambda r: (0, 0)),
            pl.BlockSpec((1, _HIDDEN), lambda r: (0, 0)),
        ],
        out_specs=[
            pl.BlockSpec((_MMBLK, _HIDDEN), lambda r: (r, 0)),
            pl.BlockSpec((_MMBLK, _HIDDEN), lambda r: (r, 0)),
        ],
        out_shape=[
            jax.ShapeDtypeStruct((_N, _HIDDEN), jnp.float32),
            jax.ShapeDtypeStruct((_N, _HIDDEN), jnp.float32),
        ],
    )(x, W1p, b1.reshape(1, _HIDDEN), W2, b2.reshape(1, _HIDDEN))


def _knn_body(pd_ref, pd2_ref, put_ref, pu2_ref, nbr_ref):
    mm = jnp.dot(pd_ref[...], put_ref[...], preferred_element_type=jnp.float32)
    d2 = (pd2_ref[...] + pu2_ref[...]) - 2.0 * mm
    col = lax.broadcasted_iota(jnp.int32, (_KNNBLK, _NUPPAD), 1)
    big = jnp.int32(2**30)
    inf = jnp.float32(jnp.inf)
    for k in range(_M):
        m = jnp.min(d2, axis=1, keepdims=True)
        amin = jnp.min(jnp.where(d2 == m, col, big), axis=1, keepdims=True)
        nbr_ref[:, k : k + 1] = amin
        d2 = jnp.where(col == amin, inf, d2)


def _knn(pd, pd2, put, pu2):
    grid = _NDOWNPAD // _KNNBLK
    return pl.pallas_call(
        _knn_body,
        grid=(grid,),
        in_specs=[
            pl.BlockSpec((_KNNBLK, 8), lambda r: (r, 0)),
            pl.BlockSpec((_KNNBLK, 1), lambda r: (r, 0)),
            pl.BlockSpec((8, _NUPPAD), lambda r: (0, 0)),
            pl.BlockSpec((1, _NUPPAD), lambda r: (0, 0)),
        ],
        out_specs=pl.BlockSpec((_KNNBLK, 8), lambda r: (r, 0)),
        out_shape=jax.ShapeDtypeStruct((_NDOWNPAD, 8), jnp.int32),
    )(pd, pd2, put, pu2)


def kernel(h, s_l, W1, b1, W2, b2):
    pos = lax.stop_gradient(s_l)
    up_mask = _fps_mask(pos)

    mask_i = up_mask.astype(jnp.int32)
    csum = jnp.cumsum(mask_i)
    ar = jnp.arange(_N, dtype=jnp.int32)
    up_idx = jnp.zeros((_NUP,), jnp.int32).at[
        jnp.where(up_mask, csum - 1, _NUP)].set(ar, mode="drop")
    down_idx = jnp.zeros((_NDOWN,), jnp.int32).at[
        jnp.where(up_mask, _NDOWN, ar - csum)].set(ar, mode="drop")

    pd = pos[down_idx]
    pu = pos[up_idx]
    pd2 = jnp.sum(pd**2, axis=1)[:, None]
    pu2 = jnp.sum(pu**2, axis=1)[None, :]

    pdp = jnp.zeros((_NDOWNPAD, 8), jnp.float32).at[:_NDOWN, :3].set(pd)
    pd2p = jnp.zeros((_NDOWNPAD, 1), jnp.float32).at[:_NDOWN].set(pd2)
    putp = jnp.zeros((8, _NUPPAD), jnp.float32).at[:3, :_NUP].set(pu.T)
    pu2p = jnp.full((1, _NUPPAD), jnp.inf, jnp.float32).at[:, :_NUP].set(pu2)
    nbr = _knn(pdp, pd2p, putp, pu2p)[:_NDOWN, :_M]

    j = up_idx[nbr.reshape(-1)]
    i = jnp.repeat(down_idx, _M)

    x = jnp.zeros((_N, _XCOLS), jnp.float32)
    x = x.at[:, :_HIDDEN].set(h).at[:, _HIDDEN : _HIDDEN + 3].set(s_l)
    W1p = jnp.zeros((_XCOLS, _HIDDEN), jnp.float32).at[: _HIDDEN + 3, :].set(W1)
    feats, G = _dense_feats_g(x, W1p, b1, W2, b2)

    Gd = G[down_idx]
    Gu = G[up_idx]
    P = jax.ops.segment_max(jnp.repeat(Gd, _M, axis=0), nbr.reshape(-1),
                            num_segments=_NUP)
    aggu = jax.nn.relu((P - Gu) + b2)
    out = feats.at[up_idx].add(aggu)
    return out, up_mask, i, j


# speedup vs baseline: 8.6403x; 1.2882x over previous
import functools

import jax
import jax.numpy as jnp
import numpy as np
from jax import lax
from jax.experimental import pallas as pl
from jax.experimental.pallas import tpu as pltpu

_N = 10000
_HIDDEN = 256
_M = 5
_NUP = 2500
_NDOWN = _N - _NUP
_ROWS, _COLS = 8, 1280
_NPAD = _ROWS * _COLS


def _fps_mask_body(px_ref, py_ref, pz_ref, xs_ref, ys_ref, zs_ref,
                   mask_ref, dists_ref):
    px = px_ref[...]
    py = py_ref[...]
    pz = pz_ref[...]
    row = lax.broadcasted_iota(jnp.int32, (_ROWS, _COLS), 0)
    col = lax.broadcasted_iota(jnp.int32, (_ROWS, _COLS), 1)
    gidx = row * _COLS + col
    gidxf = gidx.astype(jnp.float32)
    valid = gidx < _N
    neg_inf = jnp.float32(-jnp.inf)
    dists_ref[...] = jnp.where(valid, jnp.float32(1e10), neg_inf)
    mask_ref[...] = (gidx == 0).astype(jnp.int32)

    def body(_, carry):
        x0, y0, z0 = carry
        dx = px - x0
        dy = py - y0
        dz = pz - z0
        d = (dx * dx + dy * dy) + dz * dz
        dists = jnp.minimum(dists_ref[...], d)
        dists_ref[...] = dists
        m = jnp.max(dists, axis=(0, 1), keepdims=True)
        sel = jnp.where(dists == m, gidxf, jnp.float32(3e7))
        folded = sel[:, 0:128]
        for c in range(1, _COLS // 128):
            folded = jnp.minimum(folded, sel[:, c * 128 : (c + 1) * 128])
        nxtf = jnp.min(folded)
        nxt = nxtf.astype(jnp.int32)
        mask_ref[...] = jnp.maximum(mask_ref[...],
                                    (gidx == nxt).astype(jnp.int32))
        return (xs_ref[nxt], ys_ref[nxt], zs_ref[nxt])

    lax.fori_loop(1, _NUP, body, (xs_ref[0], ys_ref[0], zs_ref[0]))


def _fps_mask(pos):
    def pad(c):
        return jnp.zeros((_NPAD,), jnp.float32).at[:_N].set(c).reshape(_ROWS, _COLS)

    pxp, pyp, pzp = pad(pos[:, 0]), pad(pos[:, 1]), pad(pos[:, 2])
    xs = jnp.zeros((_NPAD,), jnp.float32).at[:_N].set(pos[:, 0])
    ys = jnp.zeros((_NPAD,), jnp.float32).at[:_N].set(pos[:, 1])
    zs = jnp.zeros((_NPAD,), jnp.float32).at[:_N].set(pos[:, 2])
    sspec = pl.BlockSpec(memory_space=pltpu.SMEM)
    mask = pl.pallas_call(
        _fps_mask_body,
        in_specs=[pl.BlockSpec((_ROWS, _COLS), lambda: (0, 0))] * 3
        + [sspec] * 3,
        out_specs=pl.BlockSpec((_ROWS, _COLS), lambda: (0, 0)),
        out_shape=jax.ShapeDtypeStruct((_ROWS, _COLS), jnp.int32),
        scratch_shapes=[pltpu.VMEM((_ROWS, _COLS), jnp.float32)],
    )(pxp, pyp, pzp, xs, ys, zs)
    return mask.reshape(-1)[:_N].astype(bool)


_XCOLS = 384
_MMBLK = 1000
_KNNBLK = 768
_NDOWNPAD = 7680
_NUPPAD = 2560


def _dense_body(x_ref, w1_ref, b1_ref, w2_ref, b2_ref, feats_ref, g_ref):
    feats = jnp.dot(x_ref[...], w1_ref[...],
                    preferred_element_type=jnp.float32) + b1_ref[...]
    feats_ref[...] = feats
    g_ref[...] = jnp.dot(feats, w2_ref[...],
                         preferred_element_type=jnp.float32)


def _dense_feats_g(x, W1p, b1, W2, b2):
    grid = _N // _MMBLK
    return pl.pallas_call(
        _dense_body,
        grid=(grid,),
        in_specs=[
            pl.BlockSpec((_MMBLK, _XCOLS), lambda r: (r, 0)),
            pl.BlockSpec((_XCOLS, _HIDDEN), lambda r: (0, 0)),
            pl.BlockSpec((1, _HIDDEN), lambda r: (0, 0)),
            pl.BlockSpec((_HIDDEN, _HIDDEN), lambda r: (0, 0)),
            pl.BlockSpec((1, _HIDDEN), lambda r: (0, 0)),
        ],
        out_specs=[
            pl.BlockSpec((_MMBLK, _HIDDEN), lambda r: (r, 0)),
            pl.BlockSpec((_MMBLK, _HIDDEN), lambda r: (r, 0)),
        ],
        out_shape=[
            jax.ShapeDtypeStruct((_N, _HIDDEN), jnp.float32),
            jax.ShapeDtypeStruct((_N, _HIDDEN), jnp.float32),
        ],
    )(x, W1p, b1.reshape(1, _HIDDEN), W2, b2.reshape(1, _HIDDEN))


def _knn_body(pd_ref, pd2_ref, put_ref, pu2_ref, nbr_ref):
    mm = jnp.dot(pd_ref[...], put_ref[...], preferred_element_type=jnp.float32)
    d2 = (pd2_ref[...] + pu2_ref[...]) - 2.0 * mm
    col = lax.broadcasted_iota(jnp.int32, (_KNNBLK, _NUPPAD), 1)
    big = jnp.int32(2**30)
    inf = jnp.float32(jnp.inf)
    for k in range(_M):
        m = jnp.min(d2, axis=1, keepdims=True)
        amin = jnp.min(jnp.where(d2 == m, col, big), axis=1, keepdims=True)
        nbr_ref[:, k : k + 1] = amin
        d2 = jnp.where(col == amin, inf, d2)


def _knn(pd, pd2, put, pu2):
    grid = _NDOWNPAD // _KNNBLK
    return pl.pallas_call(
        _knn_body,
        grid=(grid,),
        in_specs=[
            pl.BlockSpec((_KNNBLK, 8), lambda r: (r, 0)),
            pl.BlockSpec((_KNNBLK, 1), lambda r: (r, 0)),
            pl.BlockSpec((8, _NUPPAD), lambda r: (0, 0)),
            pl.BlockSpec((1, _NUPPAD), lambda r: (0, 0)),
        ],
        out_specs=pl.BlockSpec((_KNNBLK, 8), lambda r: (r, 0)),
        out_shape=jax.ShapeDtypeStruct((_NDOWNPAD, 8), jnp.int32),
    )(pd, pd2, put, pu2)


def kernel(h, s_l, W1, b1, W2, b2):
    pos = lax.stop_gradient(s_l)
    up_mask = _fps_mask(pos)

    mask_i = up_mask.astype(jnp.int32)
    csum = jnp.cumsum(mask_i)
    ar = jnp.arange(_N, dtype=jnp.int32)
    up_idx = jnp.zeros((_NUP,), jnp.int32).at[
        jnp.where(up_mask, csum - 1, _NUP)].set(ar, mode="drop")
    down_idx = jnp.zeros((_NDOWN,), jnp.int32).at[
        jnp.where(up_mask, _NDOWN, ar - csum)].set(ar, mode="drop")

    pd = pos[down_idx]
    pu = pos[up_idx]
    pd2 = jnp.sum(pd**2, axis=1)[:, None]
    pu2 = jnp.sum(pu**2, axis=1)[None, :]

    pdp = jnp.zeros((_NDOWNPAD, 8), jnp.float32).at[:_NDOWN, :3].set(pd)
    pd2p = jnp.zeros((_NDOWNPAD, 1), jnp.float32).at[:_NDOWN].set(pd2)
    putp = jnp.zeros((8, _NUPPAD), jnp.float32).at[:3, :_NUP].set(pu.T)
    pu2p = jnp.full((1, _NUPPAD), jnp.inf, jnp.float32).at[:, :_NUP].set(pu2)
    nbr = _knn(pdp, pd2p, putp, pu2p)[:_NDOWN, :_M]

    j = up_idx[nbr.reshape(-1)]
    i = jnp.repeat(down_idx, _M)

    x = jnp.zeros((_N, _XCOLS), jnp.float32)
    x = x.at[:, :_HIDDEN].set(h).at[:, _HIDDEN : _HIDDEN + 3].set(s_l)
    W1p = jnp.zeros((_XCOLS, _HIDDEN), jnp.float32).at[: _HIDDEN + 3, :].set(W1)
    feats, G = _dense_feats_g(x, W1p, b1, W2, b2)

    Gd = G[down_idx]
    Gu = G[up_idx]
    P = jax.ops.segment_max(jnp.repeat(Gd, _M, axis=0), nbr.reshape(-1),
                            num_segments=_NUP)
    aggu = jax.nn.relu((P - Gu) + b2)
    out = feats.at[up_idx].add(aggu)
    return out, up_mask, i, j


# speedup vs baseline: 9.7340x; 1.1266x over previous
import functools

import jax
import jax.numpy as jnp
import numpy as np
from jax import lax
from jax.experimental import pallas as pl
from jax.experimental.pallas import tpu as pltpu
from jax.experimental.pallas import tpu_sc as plsc

_N = 10000
_HIDDEN = 256
_M = 5
_NUP = 2500
_NDOWN = _N - _NUP
_ROWS, _COLS = 8, 1280
_NPAD = _ROWS * _COLS


def _fps_mask_body(px_ref, py_ref, pz_ref, xs_ref, ys_ref, zs_ref,
                   mask_ref, dists_ref):
    px = px_ref[...]
    py = py_ref[...]
    pz = pz_ref[...]
    row = lax.broadcasted_iota(jnp.int32, (_ROWS, _COLS), 0)
    col = lax.broadcasted_iota(jnp.int32, (_ROWS, _COLS), 1)
    gidx = row * _COLS + col
    gidxf = gidx.astype(jnp.float32)
    valid = gidx < _N
    neg_inf = jnp.float32(-jnp.inf)
    dists_ref[...] = jnp.where(valid, jnp.float32(1e10), neg_inf)
    mask_ref[...] = (gidx == 0).astype(jnp.int32)

    def body(_, carry):
        x0, y0, z0 = carry
        dx = px - x0
        dy = py - y0
        dz = pz - z0
        d = (dx * dx + dy * dy) + dz * dz
        dists = jnp.minimum(dists_ref[...], d)
        dists_ref[...] = dists
        m = jnp.max(dists, axis=(0, 1), keepdims=True)
        sel = jnp.where(dists == m, gidxf, jnp.float32(3e7))
        folded = sel[:, 0:128]
        for c in range(1, _COLS // 128):
            folded = jnp.minimum(folded, sel[:, c * 128 : (c + 1) * 128])
        nxtf = jnp.min(folded)
        nxt = nxtf.astype(jnp.int32)
        mask_ref[...] = jnp.maximum(mask_ref[...],
                                    (gidx == nxt).astype(jnp.int32))
        return (xs_ref[nxt], ys_ref[nxt], zs_ref[nxt])

    lax.fori_loop(1, _NUP, body, (xs_ref[0], ys_ref[0], zs_ref[0]))


def _fps_mask(pos):
    def pad(c):
        return jnp.zeros((_NPAD,), jnp.float32).at[:_N].set(c).reshape(_ROWS, _COLS)

    pxp, pyp, pzp = pad(pos[:, 0]), pad(pos[:, 1]), pad(pos[:, 2])
    xs = jnp.zeros((_NPAD,), jnp.float32).at[:_N].set(pos[:, 0])
    ys = jnp.zeros((_NPAD,), jnp.float32).at[:_N].set(pos[:, 1])
    zs = jnp.zeros((_NPAD,), jnp.float32).at[:_N].set(pos[:, 2])
    sspec = pl.BlockSpec(memory_space=pltpu.SMEM)
    mask = pl.pallas_call(
        _fps_mask_body,
        in_specs=[pl.BlockSpec((_ROWS, _COLS), lambda: (0, 0))] * 3
        + [sspec] * 3,
        out_specs=pl.BlockSpec((_ROWS, _COLS), lambda: (0, 0)),
        out_shape=jax.ShapeDtypeStruct((_ROWS, _COLS), jnp.int32),
        scratch_shapes=[pltpu.VMEM((_ROWS, _COLS), jnp.float32)],
    )(pxp, pyp, pzp, xs, ys, zs)
    return mask.reshape(-1)[:_N].astype(bool)


_XCOLS = 384
_MMBLK = 1000
_KNNBLK = 768
_NDOWNPAD = 7680
_NUPPAD = 2560


def _dense_body(x_ref, w1_ref, b1_ref, w2_ref, b2_ref, feats_ref, g_ref):
    feats = jnp.dot(x_ref[...], w1_ref[...],
                    preferred_element_type=jnp.float32) + b1_ref[...]
    feats_ref[...] = feats
    g_ref[...] = jnp.dot(feats, w2_ref[...],
                         preferred_element_type=jnp.float32)


def _dense_feats_g(x, W1p, b1, W2, b2):
    grid = _N // _MMBLK
    return pl.pallas_call(
        _dense_body,
        grid=(grid,),
        in_specs=[
            pl.BlockSpec((_MMBLK, _XCOLS), lambda r: (r, 0)),
            pl.BlockSpec((_XCOLS, _HIDDEN), lambda r: (0, 0)),
            pl.BlockSpec((1, _HIDDEN), lambda r: (0, 0)),
            pl.BlockSpec((_HIDDEN, _HIDDEN), lambda r: (0, 0)),
            pl.BlockSpec((1, _HIDDEN), lambda r: (0, 0)),
        ],
        out_specs=[
            pl.BlockSpec((_MMBLK, _HIDDEN), lambda r: (r, 0)),
            pl.BlockSpec((_MMBLK, _HIDDEN), lambda r: (r, 0)),
        ],
        out_shape=[
            jax.ShapeDtypeStruct((_N, _HIDDEN), jnp.float32),
            jax.ShapeDtypeStruct((_N, _HIDDEN), jnp.float32),
        ],
    )(x, W1p, b1.reshape(1, _HIDDEN), W2, b2.reshape(1, _HIDDEN))


def _knn_body(pd_ref, pd2_ref, put_ref, pu2_ref, nbr_ref):
    mm = jnp.dot(pd_ref[...], put_ref[...], preferred_element_type=jnp.float32)
    d2 = (pd2_ref[...] + pu2_ref[...]) - 2.0 * mm
    col = lax.broadcasted_iota(jnp.int32, (_KNNBLK, _NUPPAD), 1)
    big = jnp.int32(2**30)
    inf = jnp.float32(jnp.inf)
    for k in range(_M):
        m = jnp.min(d2, axis=1, keepdims=True)
        amin = jnp.min(jnp.where(d2 == m, col, big), axis=1, keepdims=True)
        nbr_ref[:, k : k + 1] = amin
        d2 = jnp.where(col == amin, inf, d2)


def _knn(pd, pd2, put, pu2):
    grid = _NDOWNPAD // _KNNBLK
    return pl.pallas_call(
        _knn_body,
        grid=(grid,),
        in_specs=[
            pl.BlockSpec((_KNNBLK, 8), lambda r: (r, 0)),
            pl.BlockSpec((_KNNBLK, 1), lambda r: (r, 0)),
            pl.BlockSpec((8, _NUPPAD), lambda r: (0, 0)),
            pl.BlockSpec((1, _NUPPAD), lambda r: (0, 0)),
        ],
        out_specs=pl.BlockSpec((_KNNBLK, 8), lambda r: (r, 0)),
        out_shape=jax.ShapeDtypeStruct((_NDOWNPAD, 8), jnp.int32),
    )(pd, pd2, put, pu2)


_NW = 32
_UPW = 80
_EPAD = 37888
_EPW = _EPAD // _NW
_DCH = 48
_NROWS = 10016
_DUMP = 10008


def _edge_body(g_hbm, f_hbm, p_hbm, up_hbm, dn_hbm, nbr_hbm, b2_hbm,
               out_hbm, j_hbm,
               b2_v, nbrj_v, jidx_v, jbuf_v, p_v, unl_v, gu_v, fu_v,
               dnl_v, dnrow_v, sem):
    wid = lax.axis_index("s") * 2 + lax.axis_index("c")
    ubase = wid * _UPW

    pltpu.sync_copy(b2_hbm, b2_v)

    ebase = wid * _EPW
    pltpu.sync_copy(nbr_hbm.at[pl.ds(ebase, _EPW)], nbrj_v.at[pl.ds(0, _EPW)])
    pad_nb = jnp.full((16,), 2559, jnp.int32)
    for t in range(_EPW // 16, 1280 // 16):
        nbrj_v[pl.ds(t * 16, 16)] = pad_nb
    for cj in range(1280 // 128):
        for t in range(8):
            jidx_v[pl.ds(t * 16, 16)] = nbrj_v[pl.ds(cj * 128 + t * 16, 16)]
        pltpu.async_copy(up_hbm.at[jidx_v], jbuf_v, sem).wait()
        wlen = min(128, _EPW - cj * 128)
        if wlen > 0:
            pltpu.sync_copy(jbuf_v.at[pl.ds(0, wlen)],
                            j_hbm.at[pl.ds(ebase + cj * 128, wlen)])

    pltpu.sync_copy(p_hbm.at[pl.ds(ubase, _UPW)], p_v)
    pltpu.sync_copy(up_hbm.at[pl.ds(ubase, _UPW)], unl_v)
    pltpu.async_copy(g_hbm.at[unl_v], gu_v, sem).wait()
    pltpu.async_copy(f_hbm.at[unl_v], fu_v, sem).wait()
    fzero = jnp.full((16,), 0.0, jnp.float32)

    def upout(q, carry):
        for cc in range(16):
            pv = p_v[q, pl.ds(cc * 16, 16)]
            gv = gu_v[q, pl.ds(cc * 16, 16)]
            bv = b2_v[pl.ds(cc * 16, 16)]
            agg = jnp.maximum((pv - gv) + bv, fzero)
            fu_v[q, pl.ds(cc * 16, 16)] = fu_v[q, pl.ds(cc * 16, 16)] + agg
        return carry

    lax.fori_loop(0, _UPW, upout, 0)
    pltpu.async_copy(fu_v, out_hbm.at[unl_v], sem).wait()

    dbase = wid * (7680 // _NW)
    for c in range(240 // _DCH):
        pltpu.sync_copy(dn_hbm.at[pl.ds(dbase + c * _DCH, _DCH)], dnl_v)
        pltpu.async_copy(f_hbm.at[dnl_v], dnrow_v, sem).wait()
        pltpu.async_copy(dnrow_v, out_hbm.at[dnl_v], sem).wait()


def _edge_aggregate(G, feats, P, up_idx, dn_idx, nbrflat, b2):
    mesh = plsc.VectorSubcoreMesh(core_axis_name="c", subcore_axis_name="s")
    f = pl.kernel(
        _edge_body,
        out_type=[
            jax.ShapeDtypeStruct((_NROWS, _HIDDEN), jnp.float32),
            jax.ShapeDtypeStruct((_EPAD,), jnp.int32),
        ],
        mesh=mesh,
        scratch_types=[
            pltpu.VMEM((_HIDDEN,), jnp.float32),
            pltpu.VMEM((1280,), jnp.int32),
            pltpu.VMEM((128,), jnp.int32),
            pltpu.VMEM((128,), jnp.int32),
            pltpu.VMEM((_UPW, _HIDDEN), jnp.float32),
            pltpu.VMEM((_UPW,), jnp.int32),
            pltpu.VMEM((_UPW, _HIDDEN), jnp.float32),
            pltpu.VMEM((_UPW, _HIDDEN), jnp.float32),
            pltpu.VMEM((_DCH,), jnp.int32),
            pltpu.VMEM((_DCH, _HIDDEN), jnp.float32),
            pltpu.SemaphoreType.DMA,
        ],
    )
    return f(G, feats, P, up_idx, dn_idx, nbrflat, b2)


def kernel(h, s_l, W1, b1, W2, b2):
    pos = lax.stop_gradient(s_l)
    up_mask = _fps_mask(pos)

    mask_i = up_mask.astype(jnp.int32)
    csum = jnp.cumsum(mask_i)
    ar = jnp.arange(_N, dtype=jnp.int32)
    up_idx = jnp.zeros((_NUP,), jnp.int32).at[
        jnp.where(up_mask, csum - 1, _NUP)].set(ar, mode="drop")
    down_idx = jnp.zeros((_NDOWN,), jnp.int32).at[
        jnp.where(up_mask, _NDOWN, ar - csum)].set(ar, mode="drop")

    pd = pos[down_idx]
    pu = pos[up_idx]
    pd2 = jnp.sum(pd**2, axis=1)[:, None]
    pu2 = jnp.sum(pu**2, axis=1)[None, :]

    pdp = jnp.zeros((_NDOWNPAD, 8), jnp.float32).at[:_NDOWN, :3].set(pd)
    pd2p = jnp.zeros((_NDOWNPAD, 1), jnp.float32).at[:_NDOWN].set(pd2)
    putp = jnp.zeros((8, _NUPPAD), jnp.float32).at[:3, :_NUP].set(pu.T)
    pu2p = jnp.full((1, _NUPPAD), jnp.inf, jnp.float32).at[:, :_NUP].set(pu2)
    nbr = _knn(pdp, pd2p, putp, pu2p)[:_NDOWN, :_M]

    i = jnp.repeat(down_idx, _M)

    x = jnp.zeros((_N, _XCOLS), jnp.float32)
    x = x.at[:, :_HIDDEN].set(h).at[:, _HIDDEN : _HIDDEN + 3].set(s_l)
    W1p = jnp.zeros((_XCOLS, _HIDDEN), jnp.float32).at[: _HIDDEN + 3, :].set(W1)
    feats, G = _dense_feats_g(x, W1p, b1, W2, b2)

    up_pad = jnp.full((2560,), _DUMP, jnp.int32).at[:_NUP].set(up_idx)
    dn_pad = jnp.full((7680,), _DUMP, jnp.int32).at[:_NDOWN].set(down_idx)
    nbrflat = jnp.full((_EPAD,), 2559, jnp.int32).at[: _NDOWN * _M].set(
        nbr.reshape(-1))
    Gp = jnp.zeros((_NROWS, _HIDDEN), jnp.float32).at[:_N].set(G)
    Fp = jnp.zeros((_NROWS, _HIDDEN), jnp.float32).at[:_N].set(feats)
    Gd = G[down_idx]
    P = jax.ops.segment_max(jnp.repeat(Gd, _M, axis=0), nbr.reshape(-1),
                            num_segments=_NUP)
    Pp = jnp.full((2560, _HIDDEN), -jnp.inf, jnp.float32).at[:_NUP].set(P)
    out_full, j_full = _edge_aggregate(Gp, Fp, Pp, up_pad, dn_pad, nbrflat, b2)
    out = out_full[:_N]
    j = j_full[: _NDOWN * _M]
    return out, up_mask, i, j
